# baseline (speedup 1.0000x reference)
import jax
import jax.numpy as jnp
from jax import lax
from jax.experimental import pallas as pl
from jax.experimental.pallas import tpu as pltpu


def _mr_kernel(eidx_ref, x_ref, w_ref, bias_ref, o_ref, *, kc, num_k):
    bt, c, n = x_ref.shape
    one = jnp.bfloat16(1.0)
    none_ = jnp.bfloat16(-1.0)
    zero = jnp.bfloat16(0.0)

    io = lax.broadcasted_iota(jnp.int32, (n, kc * n), 0)
    io = jnp.where(io < 256, io + 1, 255 - io).astype(jnp.bfloat16)

    w = w_ref[...]
    bias = bias_ref[...]

    for b in range(bt):
        x_b = x_ref[b]
        tj = jnp.transpose(eidx_ref[0, b])
        ti = jnp.transpose(eidx_ref[1, b])
        eq = tj == ti
        fj = jnp.where(tj < 256, tj + 1, 255 - tj)
        fi = jnp.where(ti < 256, ti + 1, 255 - ti)
        fj = jnp.where(eq, 384, fj).astype(jnp.bfloat16)
        fi = jnp.where(eq, -384, fi).astype(jnp.bfloat16)

        rel = None
        for ch in range(num_k // kc):
            i_j = jnp.concatenate(
                [fj[k:k + 1, :] for k in range(ch * kc, (ch + 1) * kc)], axis=1)
            i_i = jnp.concatenate(
                [fi[k:k + 1, :] for k in range(ch * kc, (ch + 1) * kc)], axis=1)
            d = jnp.where(io == i_j, one,
                          jnp.where(io == i_i, none_, zero))
            g = jnp.dot(x_b, d, preferred_element_type=jnp.float32)
            m = g[:, :n]
            for kk in range(1, kc):
                m = jnp.maximum(m, g[:, kk * n:(kk + 1) * n])
            rel = m if rel is None else jnp.maximum(rel, m)

        xr = jnp.concatenate([x_b, rel.astype(jnp.bfloat16)], axis=0)
        out = jnp.dot(w, xr, preferred_element_type=jnp.float32) + bias
        o_ref[b] = jnp.maximum(out, 0.0)


def kernel(x, edge_index, W, b):
    B, C, N, _ = x.shape
    K = edge_index.shape[-1]
    Cout = W.shape[0]

    n_pad = ((N + 127) // 128) * 128
    bt = 4
    b_pad = ((B + bt - 1) // bt) * bt
    kc = 4 if K % 4 == 0 else 1

    x_p = jnp.pad(x[..., 0], ((0, b_pad - B), (0, 0), (0, n_pad - N)))
    x_p = x_p.astype(jnp.bfloat16)

    eidx = jnp.pad(edge_index.astype(jnp.int32),
                   ((0, 0), (0, b_pad - B), (0, n_pad - N), (0, 0)))

    w2 = jnp.concatenate([W[:, 0::2], W[:, 1::2]], axis=1).astype(jnp.bfloat16)
    bias = b.reshape(Cout, 1).astype(jnp.float32)

    out = pl.pallas_call(
        lambda *refs: _mr_kernel(*refs, kc=kc, num_k=K),
        out_shape=jax.ShapeDtypeStruct((b_pad, Cout, n_pad), jnp.float32),
        grid=(b_pad // bt,),
        in_specs=[
            pl.BlockSpec((2, bt, n_pad, K), lambda i: (0, i, 0, 0)),
            pl.BlockSpec((bt, C, n_pad), lambda i: (i, 0, 0)),
            pl.BlockSpec((Cout, 2 * C), lambda i: (0, 0)),
            pl.BlockSpec((Cout, 1), lambda i: (0, 0)),
        ],
        out_specs=pl.BlockSpec((bt, Cout, n_pad), lambda i: (i, 0, 0)),
        compiler_params=pltpu.CompilerParams(dimension_semantics=("parallel",)),
    )(eidx, x_p, w2, bias)

    return out[:B, :, :N][..., None, None]


# speedup vs baseline: 1.0515x; 1.0515x over previous
import jax
import jax.numpy as jnp
from jax import lax
from jax.experimental import pallas as pl
from jax.experimental.pallas import tpu as pltpu


def _mr_kernel(idx_j_ref, idx_i_ref, x_ref, w_ref, bias_ref, o_ref, *, kc, num_k):
    bt, c, n = x_ref.shape
    one = jnp.bfloat16(1.0)
    none_ = jnp.bfloat16(-1.0)
    zero = jnp.bfloat16(0.0)

    io = lax.broadcasted_iota(jnp.int32, (n, kc * n), 0)
    io = jnp.where(io < 256, io + 1, 255 - io).astype(jnp.bfloat16)
    w = w_ref[...]
    bias = bias_ref[...]

    for b in range(bt):
        x_b = x_ref[b]
        rel = None
        for ch in range(num_k // kc):
            lo = ch * kc * n
            i_j = idx_j_ref[b, :, lo:lo + kc * n]
            i_i = idx_i_ref[b, :, lo:lo + kc * n]
            d = jnp.where(io == i_j, one,
                          jnp.where(io == i_i, none_, zero))
            g = jnp.dot(x_b, d, preferred_element_type=jnp.float32)
            m = g[:, :n]
            for kk in range(1, kc):
                m = jnp.maximum(m, g[:, kk * n:(kk + 1) * n])
            rel = m if rel is None else jnp.maximum(rel, m)

        xr = jnp.concatenate([x_b, rel.astype(jnp.bfloat16)], axis=0)
        out = jnp.dot(w, xr, preferred_element_type=jnp.float32) + bias
        o_ref[b] = jnp.maximum(out, 0.0)


def kernel(x, edge_index, W, b):
    B, C, N, _ = x.shape
    K = edge_index.shape[-1]
    Cout = W.shape[0]

    n_pad = ((N + 127) // 128) * 128
    bt = 8
    b_pad = ((B + bt - 1) // bt) * bt
    kc = 4 if K % 4 == 0 else 1

    x_p = jnp.pad(x[..., 0], ((0, b_pad - B), (0, 0), (0, n_pad - N)))
    x_p = x_p.astype(jnp.bfloat16)

    idx = jnp.transpose(edge_index.astype(jnp.int32), (0, 1, 3, 2))
    idx = jnp.pad(idx, ((0, 0), (0, b_pad - B), (0, 0), (0, n_pad - N)))
    eq = idx[0] == idx[1]
    idx = jnp.where(idx < 256, idx + 1, 255 - idx).astype(jnp.bfloat16)
    idx_j = jnp.where(eq, jnp.bfloat16(384), idx[0]).reshape(b_pad, 1, K * n_pad)
    idx_i = jnp.where(eq, jnp.bfloat16(-384), idx[1]).reshape(b_pad, 1, K * n_pad)


    w2 = jnp.concatenate([W[:, 0::2], W[:, 1::2]], axis=1).astype(jnp.bfloat16)
    bias = b.reshape(Cout, 1).astype(jnp.float32)

    out = pl.pallas_call(
        lambda *refs: _mr_kernel(*refs, kc=kc, num_k=K),
        out_shape=jax.ShapeDtypeStruct((b_pad, Cout, n_pad), jnp.float32),
        grid=(b_pad // bt,),
        in_specs=[
            pl.BlockSpec((bt, 1, K * n_pad), lambda i: (i, 0, 0)),
            pl.BlockSpec((bt, 1, K * n_pad), lambda i: (i, 0, 0)),
            pl.BlockSpec((bt, C, n_pad), lambda i: (i, 0, 0)),
            pl.BlockSpec((Cout, 2 * C), lambda i: (0, 0)),
            pl.BlockSpec((Cout, 1), lambda i: (0, 0)),
        ],
        out_specs=pl.BlockSpec((bt, Cout, n_pad), lambda i: (i, 0, 0)),
        compiler_params=pltpu.CompilerParams(dimension_semantics=("parallel",)),
    )(idx_j, idx_i, x_p, w2, bias)

    return out[:B, :, :N][..., None, None]


# speedup vs baseline: 1.0877x; 1.0344x over previous
import jax
import jax.numpy as jnp
from jax import lax
from jax.experimental import pallas as pl
from jax.experimental.pallas import tpu as pltpu


def _mr_kernel(idx_ref, x_ref, w_ref, bias_ref, o_ref, *, kc, num_k):
    bt, c, n = x_ref.shape
    one = jnp.bfloat16(1.0)
    none_ = jnp.bfloat16(-1.0)
    zero = jnp.bfloat16(0.0)

    io = lax.broadcasted_iota(jnp.int32, (n, kc * n), 0)
    io = jnp.where(io < 256, io + 1, 255 - io).astype(jnp.bfloat16)
    w = w_ref[...]
    bias = bias_ref[...]

    for b in range(bt):
        x_b = x_ref[b].astype(jnp.bfloat16)
        rel = None
        for ch in range(num_k // kc):
            lo = ch * kc * n
            ji = pltpu.bitcast(idx_ref[b, :, lo:lo + kc * n], jnp.bfloat16)
            i_j = ji[0:1, :]
            i_i = ji[1:2, :]
            d = jnp.where(io == i_j, one,
                          jnp.where(io == i_i, none_, zero))
            g = jnp.dot(x_b, d, preferred_element_type=jnp.float32)
            m = g[:, :n]
            for kk in range(1, kc):
                m = jnp.maximum(m, g[:, kk * n:(kk + 1) * n])
            rel = m if rel is None else jnp.maximum(rel, m)

        xr = jnp.concatenate([x_b, rel.astype(jnp.bfloat16)], axis=0)
        out = jnp.dot(w, xr, preferred_element_type=jnp.float32) + bias
        o_ref[b] = jnp.maximum(out, 0.0)


def kernel(x, edge_index, W, b):
    B, C, N, _ = x.shape
    K = edge_index.shape[-1]
    Cout = W.shape[0]

    n_pad = ((N + 127) // 128) * 128
    bt = 8
    b_pad = ((B + bt - 1) // bt) * bt
    kc = 4 if K % 4 == 0 else 1

    x_p = jnp.pad(x[..., 0], ((0, b_pad - B), (0, 0), (0, n_pad - N)))

    idx = edge_index.astype(jnp.int32)
    eq = idx[0] == idx[1]
    idxm = jnp.where(idx < 256, idx + 1, 255 - idx).astype(jnp.bfloat16)
    fj = jnp.where(eq, jnp.bfloat16(384), idxm[0])
    fi = jnp.where(eq, jnp.bfloat16(-384), idxm[1])
    comb = (lax.bitcast_convert_type(fj, jnp.uint16).astype(jnp.uint32)
            | (lax.bitcast_convert_type(fi, jnp.uint16).astype(jnp.uint32) << 16)
            ).astype(jnp.int32)
    comb = jnp.transpose(comb, (0, 2, 1))
    comb = jnp.pad(comb, ((0, b_pad - B), (0, 0), (0, n_pad - N)))
    comb = comb.reshape(b_pad, 1, K * n_pad)


    w2 = jnp.concatenate([W[:, 0::2], W[:, 1::2]], axis=1).astype(jnp.bfloat16)
    bias = b.reshape(Cout, 1).astype(jnp.float32)

    out = pl.pallas_call(
        lambda *refs: _mr_kernel(*refs, kc=kc, num_k=K),
        out_shape=jax.ShapeDtypeStruct((b_pad, Cout, n_pad), jnp.float32),
        grid=(b_pad // bt,),
        in_specs=[
            pl.BlockSpec((bt, 1, K * n_pad), lambda i: (i, 0, 0)),
            pl.BlockSpec((bt, C, n_pad), lambda i: (i, 0, 0)),
            pl.BlockSpec((Cout, 2 * C), lambda i: (0, 0)),
            pl.BlockSpec((Cout, 1), lambda i: (0, 0)),
        ],
        out_specs=pl.BlockSpec((bt, Cout, n_pad), lambda i: (i, 0, 0)),
        compiler_params=pltpu.CompilerParams(dimension_semantics=("parallel",)),
    )(comb, x_p, w2, bias)

    return out[:B, :, :N][..., None, None]
